# baseline (speedup 1.0000x reference)
import jax
import jax.numpy as jnp
from jax.experimental import pallas as pl
from jax.experimental.pallas import tpu as pltpu

_A, _B, _C, _D = 16, 128, 128, 256
_EPS = 1e-6
_ROWS = _A * _B
_CH = 64
_NCH = _ROWS // _CH
_CPA = _B // _CH
_K = 3


def _wide_kernel(pe0_ref, pe1_ref, pe2_ref, out_ref, buf_ref, sem_ref):
    pe2 = pe2_ref[0:_C, :]
    pe2b = pe2.astype(jnp.bfloat16)
    n2 = jnp.sum(pe2 * pe2, axis=-1)

    def body(i, carry):
        slot = jax.lax.rem(i, _K)

        @pl.when(i >= _K)
        def _():
            pltpu.make_async_copy(
                buf_ref.at[slot],
                out_ref.at[pl.ds((i - _K) * _CH, _CH)],
                sem_ref.at[slot],
            ).wait()

        a = i // _CPA
        b0 = (i % _CPA) * _CH
        rows = pe0_ref[a, :][None, :] + pe1_ref[pl.ds(b0, _CH), :]
        n01 = jnp.sum(rows * rows, axis=-1, keepdims=True)
        dots = jax.lax.dot_general(
            rows.astype(jnp.bfloat16),
            pe2b,
            (((1,), (1,)), ((), ())),
            preferred_element_type=jnp.float32,
        )
        ssq = n01 + 2.0 * dots + n2[None, :]
        recip = 1.0 / (_EPS + jnp.sqrt(ssq) * (1.0 / 16.0))
        buf_ref[slot] = (rows[:, None, :] + pe2[None, :, :]) * recip[:, :, None]
        pltpu.make_async_copy(
            buf_ref.at[slot],
            out_ref.at[pl.ds(i * _CH, _CH)],
            sem_ref.at[slot],
        ).start()
        return carry

    jax.lax.fori_loop(0, _NCH, body, 0)

    def drain(j, carry):
        pltpu.make_async_copy(
            buf_ref.at[jax.lax.rem(j, _K)],
            out_ref.at[pl.ds(j * _CH, _CH)],
            sem_ref.at[jax.lax.rem(j, _K)],
        ).wait()
        return carry

    jax.lax.fori_loop(_NCH - _K, _NCH, drain, 0)


def kernel(pos_embed_0, pos_embed_1, pos_embed_2, axial0, axial1, axial2):
    out = pl.pallas_call(
        _wide_kernel,
        in_specs=[
            pl.BlockSpec(memory_space=pltpu.MemorySpace.VMEM),
            pl.BlockSpec(memory_space=pltpu.MemorySpace.VMEM),
            pl.BlockSpec(memory_space=pltpu.MemorySpace.VMEM),
        ],
        out_specs=pl.BlockSpec(memory_space=pltpu.MemorySpace.HBM),
        out_shape=jax.ShapeDtypeStruct((_ROWS, _C, _D), jnp.float32),
        scratch_shapes=[
            pltpu.MemorySpace.VMEM((_K, _CH, _C, _D), jnp.float32),
            pltpu.SemaphoreType.DMA((_K,)),
        ],
    )(pos_embed_0, pos_embed_1, pos_embed_2)
    return out.reshape(_A, _B, _C, _D)
